# baseline (speedup 1.0000x reference)
import functools

import jax
import jax.numpy as jnp
from jax import lax
from jax.experimental import pallas as pl
from jax.experimental.pallas import tpu as pltpu
from jax.experimental.pallas import tpu_sc as plsc

Nv_ = 100000
Nh_ = 16
BATCH_ = 4096
XMIN_ = -10.0
XMAX_ = 10.0
DX_ = (XMAX_ - XMIN_) / (Nv_ - 1)

_NC = 2
_NS = 16
_NW = _NC * _NS
_BPW = BATCH_ // _NW
_L = 16


def _rbm_body(x_hbm, b_hbm, c_hbm, w_hbm, out_hbm,
              x_v, idx_v, rows_v, t_v, b_v, c_v, out_v, sem_w, sem_b):
    wid = lax.axis_index("s") * _NC + lax.axis_index("c")
    base = wid * _BPW

    pltpu.sync_copy(x_hbm.at[pl.ds(base, _BPW)], x_v)
    pltpu.sync_copy(c_hbm, c_v)

    def idx_body(k, carry):
        off = pl.multiple_of(k * _L, _L)
        xv = x_v[pl.ds(off, _L)]
        ii = ((xv - XMIN_) / DX_).astype(jnp.int32)
        idx_v[pl.ds(off, _L)] = jnp.minimum(jnp.maximum(ii, 0), Nv_ - 1)
        return carry

    lax.fori_loop(0, _BPW // _L, idx_body, 0, unroll=False)

    cp_w = pltpu.async_copy(w_hbm.at[idx_v], rows_v, sem_w)
    cp_b = pltpu.async_copy(b_hbm.at[idx_v], b_v, sem_b)
    cp_w.wait()
    cp_b.wait()

    cv = c_v[...]
    lane16 = lax.iota(jnp.int32, _L) * Nh_

    def chunk_body(k, carry):
        row0 = pl.multiple_of(k * _L, _L)
        for i in range(_L):
            t_v[pl.ds(pl.multiple_of((row0 + i) * Nh_, _L), Nh_)] = (
                1.0 + jnp.exp(cv + rows_v[row0 + i]))
        acc = jnp.exp(b_v[pl.ds(row0, _L)])
        tbase = lane16 + row0 * Nh_
        for h in range(Nh_):
            acc = acc * plsc.load_gather(t_v, [tbase + h])
        out_v[pl.ds(row0, _L)] = acc
        return carry

    lax.fori_loop(0, _BPW // _L, chunk_body, 0, unroll=False)

    pltpu.sync_copy(out_v, out_hbm.at[pl.ds(base, _BPW)])


_rbm_sc = functools.partial(
    pl.kernel,
    out_type=jax.ShapeDtypeStruct((BATCH_,), jnp.float32),
    mesh=plsc.VectorSubcoreMesh(core_axis_name="c", subcore_axis_name="s"),
    compiler_params=pltpu.CompilerParams(needs_layout_passes=False,
                                         use_tc_tiling_on_sc=False),
    scratch_types=[
        pltpu.VMEM((_BPW,), jnp.float32),
        pltpu.VMEM((_BPW,), jnp.int32),
        pltpu.VMEM((_BPW, Nh_), jnp.float32),
        pltpu.VMEM((_BPW * Nh_,), jnp.float32),
        pltpu.VMEM((_BPW,), jnp.float32),
        pltpu.VMEM((Nh_,), jnp.float32),
        pltpu.VMEM((_BPW,), jnp.float32),
        pltpu.SemaphoreType.DMA,
        pltpu.SemaphoreType.DMA,
    ],
)(_rbm_body)


def kernel(x, b, c, w):
    return _rbm_sc(x, b, c, w)


# speedup vs baseline: 2.1976x; 2.1976x over previous
import functools

import jax
import jax.numpy as jnp
from jax import lax
from jax.experimental import pallas as pl
from jax.experimental.pallas import tpu as pltpu
from jax.experimental.pallas import tpu_sc as plsc

Nv_ = 100000
Nh_ = 16
BATCH_ = 4096
XMIN_ = -10.0
XMAX_ = 10.0
DX_ = (XMAX_ - XMIN_) / (Nv_ - 1)

_NC = 2
_NS = 16
_NW = _NC * _NS
_BPW = BATCH_ // _NW
_L = 16


def _rbm_body(x_hbm, b_hbm, c_hbm, wt_hbm, out_hbm,
              x_v, idx_v, t_v, b_v, c_v, out_v, sem_w, sem_b):
    wid = lax.axis_index("s") * _NC + lax.axis_index("c")
    base = wid * _BPW

    pltpu.sync_copy(x_hbm.at[pl.ds(base, _BPW)], x_v)
    pltpu.sync_copy(c_hbm, c_v)

    def idx_body(k, carry):
        off = pl.multiple_of(k * _L, _L)
        xv = x_v[pl.ds(off, _L)]
        ii = ((xv - XMIN_) / DX_).astype(jnp.int32)
        idx_v[pl.ds(off, _L)] = jnp.minimum(jnp.maximum(ii, 0), Nv_ - 1)
        return carry

    lax.fori_loop(0, _BPW // _L, idx_body, 0, unroll=False)

    cp_b = pltpu.async_copy(b_hbm.at[idx_v], b_v, sem_b)
    cps = [pltpu.async_copy(wt_hbm.at[h].at[idx_v], t_v.at[h], sem_w)
           for h in range(Nh_)]
    cp_b.wait()
    for cp in cps:
        cp.wait()

    chs = [plsc.load_gather(c_v, [jnp.full((_L,), h, jnp.int32)])
           for h in range(Nh_)]

    def chunk_body(k, carry):
        off = pl.multiple_of(k * _L, _L)
        acc = jnp.exp(b_v[pl.ds(off, _L)])
        for h in range(Nh_):
            acc = acc * (1.0 + jnp.exp(chs[h] + t_v[h, pl.ds(off, _L)]))
        out_v[pl.ds(off, _L)] = acc
        return carry

    lax.fori_loop(0, _BPW // _L, chunk_body, 0, unroll=False)

    pltpu.sync_copy(out_v, out_hbm.at[pl.ds(base, _BPW)])


_rbm_sc = functools.partial(
    pl.kernel,
    out_type=jax.ShapeDtypeStruct((BATCH_,), jnp.float32),
    mesh=plsc.VectorSubcoreMesh(core_axis_name="c", subcore_axis_name="s"),
    compiler_params=pltpu.CompilerParams(needs_layout_passes=False,
                                         use_tc_tiling_on_sc=False),
    scratch_types=[
        pltpu.VMEM((_BPW,), jnp.float32),
        pltpu.VMEM((_BPW,), jnp.int32),
        pltpu.VMEM((Nh_, _BPW), jnp.float32),
        pltpu.VMEM((_BPW,), jnp.float32),
        pltpu.VMEM((Nh_,), jnp.float32),
        pltpu.VMEM((_BPW,), jnp.float32),
        pltpu.SemaphoreType.DMA,
        pltpu.SemaphoreType.DMA,
    ],
)(_rbm_body)


def kernel(x, b, c, w):
    return _rbm_sc(x, b, c, w.T)
